# Initial kernel scaffold; baseline (speedup 1.0000x reference)
import jax
import jax.numpy as jnp
from jax.experimental import pallas as pl


def kernel(out_features):
    raise NotImplementedError("write your pallas kernel here")


# speedup vs baseline: 10.7111x; 10.7111x over previous
import functools

import jax
import jax.numpy as jnp
from jax.experimental import pallas as pl
from jax.experimental.pallas import tpu as pltpu

_C = 80
_H = 128
_W = 128
_K = 100
_THRESH = 0.3
_DOWN = 4.0
_IMG = 512.0
_BIG = 10 ** 9


def _decode_body(x_ref, o_ref, s_ref, rm_ref):
    x = x_ref[0]
    hm = 1.0 / (1.0 + jnp.exp(-x[:_C]))

    zrow = jnp.zeros((_C, 1, _W), jnp.float32)
    up = jnp.concatenate([hm[:, 1:, :], zrow], axis=1)
    dn = jnp.concatenate([zrow, hm[:, :-1, :]], axis=1)
    vmx = jnp.maximum(hm, jnp.maximum(up, dn))
    zcol = jnp.zeros((_C, _H, 1), jnp.float32)
    lf = jnp.concatenate([vmx[:, :, 1:], zcol], axis=2)
    rt = jnp.concatenate([zcol, vmx[:, :, :-1]], axis=2)
    pooled = jnp.maximum(vmx, jnp.maximum(lf, rt))

    sup = jnp.where(hm == pooled, hm, 0.0)
    s_ref[...] = sup.reshape(_C * _H, _W)
    rm_ref[...] = jnp.max(sup, axis=2)

    rows_iota = (jax.lax.broadcasted_iota(jnp.int32, (_C, _H), 0) * _H
                 + jax.lax.broadcasted_iota(jnp.int32, (_C, _H), 1))
    lane_i = jax.lax.broadcasted_iota(jnp.int32, (1, _W), 1)
    lane_f = lane_i.astype(jnp.float32)

    def step(k, _):
        rm = rm_ref[...]
        m = jnp.max(rm)
        r = jnp.min(jnp.where(rm == m, rows_iota, _BIG))
        row = s_ref[pl.ds(r, 1), :]
        xcol = jnp.min(jnp.where(row == m, lane_i, _BIG))
        hit = lane_i == xcol
        new_row = jnp.where(hit, 0.0, row)
        s_ref[pl.ds(r, 1), :] = new_row
        nm = jnp.max(new_row)
        c_ = r // _H
        y_ = r - c_ * _H
        rmrow = rm_ref[pl.ds(c_, 1), :]
        rm_ref[pl.ds(c_, 1), :] = jnp.where(lane_i == y_, nm, rmrow)

        def pick(ch):
            v = x_ref[0, pl.ds(_C + ch, 1), pl.ds(y_, 1), :].reshape(1, _W)
            return jnp.sum(jnp.where(hit, v, 0.0))

        rx, ry, ww, wl = pick(0), pick(1), pick(2), pick(3)
        xs = xcol.astype(jnp.float32) + rx
        ys = y_.astype(jnp.float32) + ry
        x1 = jnp.clip((xs - ww * 0.5) * _DOWN, 0.0, _IMG)
        y1 = jnp.clip((ys - wl * 0.5) * _DOWN, 0.0, _IMG)
        x2 = jnp.clip((xs + ww * 0.5) * _DOWN, 0.0, _IMG)
        y2 = jnp.clip((ys + wl * 0.5) * _DOWN, 0.0, _IMG)
        valid = (m >= _THRESH).astype(jnp.float32)

        rowvec = jnp.where(lane_i == 0, x1,
                  jnp.where(lane_i == 1, y1,
                   jnp.where(lane_i == 2, x2,
                    jnp.where(lane_i == 3, y2,
                     jnp.where(lane_i == 4, m,
                      jnp.where(lane_i == 5, c_.astype(jnp.float32), 0.0))))))
        o_ref[0, pl.ds(k, 1), :] = rowvec * valid
        return 0

    jax.lax.fori_loop(0, _K, step, 0)


@jax.jit
def kernel(out_features):
    b = out_features.shape[0]
    padded = pl.pallas_call(
        _decode_body,
        grid=(b,),
        in_specs=[pl.BlockSpec((1, _C + 4, _H, _W), lambda i: (i, 0, 0, 0))],
        out_specs=pl.BlockSpec((1, 104, _W), lambda i: (i, 0, 0)),
        out_shape=jax.ShapeDtypeStruct((b, 104, _W), jnp.float32),
        scratch_shapes=[
            pltpu.VMEM((_C * _H, _W), jnp.float32),
            pltpu.VMEM((_C, _H), jnp.float32),
        ],
    )(out_features)
    return padded[:, :_K, :6]


# speedup vs baseline: 12.1690x; 1.1361x over previous
import functools

import jax
import jax.numpy as jnp
from jax.experimental import pallas as pl
from jax.experimental.pallas import tpu as pltpu

_B = 8
_C = 80
_G = 20
_H = 128
_W = 128
_K = 100
_THRESH = 0.3
_DOWN = 4.0
_IMG = 512.0
_BIG = 10 ** 9
_NEG = -1e30


def _body(x_hbm, o_ref, stage, sem, s_refs, rm_refs, rw_refs):
    i = pl.program_id(0)
    cp = pltpu.make_async_copy(x_hbm.at[i], stage, sem)
    cp.start()
    cp.wait()

    for g in range(_C // _G):
        xg = stage[pl.ds(g * _G, _G)]
        zrow = jnp.full((_G, 1, _W), _NEG, jnp.float32)
        up = jnp.concatenate([xg[:, 1:, :], zrow], axis=1)
        dn = jnp.concatenate([zrow, xg[:, :-1, :]], axis=1)
        vmx = jnp.maximum(xg, jnp.maximum(up, dn))
        zcol = jnp.full((_G, _H, 1), _NEG, jnp.float32)
        lf = jnp.concatenate([vmx[:, :, 1:], zcol], axis=2)
        rt = jnp.concatenate([zcol, vmx[:, :, :-1]], axis=2)
        pooled = jnp.maximum(vmx, jnp.maximum(lf, rt))
        supg = jnp.where(xg == pooled, xg, _NEG)
        rmg = jnp.max(supg, axis=2)
        supg = supg.reshape(_G * _H, _W)
        for b in range(_B):
            @pl.when(i == b)
            def _store(b=b, g=g, supg=supg, rmg=rmg):
                s_refs[b][pl.ds(g * _G * _H, _G * _H), :] = supg
                rm_refs[b][pl.ds(g * _G, _G), :] = rmg

    for b in range(_B):
        @pl.when(i == b)
        def _store_rw(b=b):
            rw_refs[b][...] = stage[pl.ds(_C, 4)].reshape(4 * _H, _W)

    rows_iota = (jax.lax.broadcasted_iota(jnp.int32, (_C, _H), 0) * _H
                 + jax.lax.broadcasted_iota(jnp.int32, (_C, _H), 1))
    lane_i = jax.lax.broadcasted_iota(jnp.int32, (1, _W), 1)

    @pl.when(i == _B - 1)
    def _extract():
        def step(k, _):
            for b in range(_B):
                rm = rm_refs[b][...]
                m = jnp.max(rm)
                r = jnp.min(jnp.where(rm == m, rows_iota, _BIG))
                row = s_refs[b][pl.ds(r, 1), :]
                xcol = jnp.min(jnp.where(row == m, lane_i, _BIG))
                hit = lane_i == xcol
                new_row = jnp.where(hit, _NEG, row)
                s_refs[b][pl.ds(r, 1), :] = new_row
                nm = jnp.max(new_row)
                c_ = r // _H
                y_ = r - c_ * _H
                rmrow = rm_refs[b][pl.ds(c_, 1), :]
                rm_refs[b][pl.ds(c_, 1), :] = jnp.where(lane_i == y_, nm, rmrow)

                def pick(ch):
                    v = rw_refs[b][pl.ds(ch * _H + y_, 1), :]
                    return jnp.sum(jnp.where(hit, v, 0.0))

                rx, ry, ww, wl = pick(0), pick(1), pick(2), pick(3)
                xs = xcol.astype(jnp.float32) + rx
                ys = y_.astype(jnp.float32) + ry
                x1 = jnp.clip((xs - ww * 0.5) * _DOWN, 0.0, _IMG)
                y1 = jnp.clip((ys - wl * 0.5) * _DOWN, 0.0, _IMG)
                x2 = jnp.clip((xs + ww * 0.5) * _DOWN, 0.0, _IMG)
                y2 = jnp.clip((ys + wl * 0.5) * _DOWN, 0.0, _IMG)

                sig_vec = 1.0 / (1.0 + jnp.exp(-jnp.where(lane_i == 4, m, 0.0)))
                score = jnp.max(jnp.where(lane_i == 4, sig_vec, -1.0))
                valid = (score >= _THRESH).astype(jnp.float32)

                rowvec = jnp.where(lane_i == 0, x1,
                          jnp.where(lane_i == 1, y1,
                           jnp.where(lane_i == 2, x2,
                            jnp.where(lane_i == 3, y2,
                             jnp.where(lane_i == 4, score,
                              jnp.where(lane_i == 5,
                                        c_.astype(jnp.float32), 0.0))))))
                o_ref[b, pl.ds(k, 1), :] = rowvec * valid
            return 0

        jax.lax.fori_loop(0, _K, step, 0)


def _wrapped(x):
    body = lambda x_hbm, o_ref, stage, sem, *rest: _body(
        x_hbm, o_ref, stage, sem,
        list(rest[:_B]), list(rest[_B:2 * _B]), list(rest[2 * _B:]))
    return pl.pallas_call(
        body,
        grid=(_B,),
        in_specs=[pl.BlockSpec(memory_space=pl.ANY)],
        out_specs=pl.BlockSpec((_B, 104, _W), lambda i: (0, 0, 0)),
        out_shape=jax.ShapeDtypeStruct((_B, 104, _W), jnp.float32),
        scratch_shapes=(
            [pltpu.VMEM((_C + 4, _H, _W), jnp.float32),
             pltpu.SemaphoreType.DMA]
            + [pltpu.VMEM((_C * _H, _W), jnp.float32) for _ in range(_B)]
            + [pltpu.VMEM((_C, _H), jnp.float32) for _ in range(_B)]
            + [pltpu.VMEM((4 * _H, _W), jnp.float32) for _ in range(_B)]
        ),
    )(x)


@jax.jit
def kernel(out_features):
    padded = _wrapped(out_features)
    return padded[:, :_K, :6]


# speedup vs baseline: 35.0463x; 2.8800x over previous
import functools

import jax
import jax.numpy as jnp
from jax.experimental import pallas as pl
from jax.experimental.pallas import tpu as pltpu

_B = 8
_C = 80
_G = 20
_H = 128
_W = 128
_K = 100
_THRESH = 0.3
_DOWN = 4.0
_IMG = 512.0
_BIG = 10 ** 9
_NEG = -1e30


def _body(x_hbm, o_ref, stage, sem, s_refs, rm_refs, rw_refs):
    i = pl.program_id(0)
    cp = pltpu.make_async_copy(x_hbm.at[i], stage, sem)
    cp.start()
    cp.wait()

    for g in range(_C // _G):
        xg = stage[pl.ds(g * _G, _G)]
        zrow = jnp.full((_G, 1, _W), _NEG, jnp.float32)
        up = jnp.concatenate([xg[:, 1:, :], zrow], axis=1)
        dn = jnp.concatenate([zrow, xg[:, :-1, :]], axis=1)
        vmx = jnp.maximum(xg, jnp.maximum(up, dn))
        zcol = jnp.full((_G, _H, 1), _NEG, jnp.float32)
        lf = jnp.concatenate([vmx[:, :, 1:], zcol], axis=2)
        rt = jnp.concatenate([zcol, vmx[:, :, :-1]], axis=2)
        pooled = jnp.maximum(vmx, jnp.maximum(lf, rt))
        supg = jnp.where(xg == pooled, xg, _NEG)
        rmg = jnp.max(supg, axis=2)
        supg = supg.reshape(_G * _H, _W)
        for b in range(_B):
            @pl.when(i == b)
            def _store(b=b, g=g, supg=supg, rmg=rmg):
                s_refs[b][pl.ds(g * _G * _H, _G * _H), :] = supg
                rm_refs[b][pl.ds(g * _G, _G), :] = rmg

    for b in range(_B):
        @pl.when(i == b)
        def _store_rw(b=b):
            rw_refs[b][...] = stage[pl.ds(_C, 4)].reshape(4 * _H, _W)

    rows_iota = (jax.lax.broadcasted_iota(jnp.int32, (_C, _H), 0) * _H
                 + jax.lax.broadcasted_iota(jnp.int32, (_C, _H), 1))
    lane_i = jax.lax.broadcasted_iota(jnp.int32, (1, _W), 1)

    @pl.when(i == _B - 1)
    def _extract():
        def step(k, _):
            B = range(_B)
            rms = [rm_refs[b][...] for b in B]
            ms = [jnp.max(rms[b]) for b in B]
            rs = [jnp.min(jnp.where(rms[b] == ms[b], rows_iota, _BIG))
                  for b in B]
            rows = [s_refs[b][pl.ds(rs[b], 1), :] for b in B]
            xcols = [jnp.min(jnp.where(rows[b] == ms[b], lane_i, _BIG))
                     for b in B]
            hits = [lane_i == xcols[b] for b in B]
            new_rows = [jnp.where(hits[b], _NEG, rows[b]) for b in B]
            for b in B:
                s_refs[b][pl.ds(rs[b], 1), :] = new_rows[b]
            nms = [jnp.max(new_rows[b]) for b in B]
            cs = [rs[b] // _H for b in B]
            ys_ = [rs[b] - cs[b] * _H for b in B]
            rmrows = [rm_refs[b][pl.ds(cs[b], 1), :] for b in B]
            for b in B:
                rm_refs[b][pl.ds(cs[b], 1), :] = jnp.where(
                    lane_i == ys_[b], nms[b], rmrows[b])

            picks = []
            for b in B:
                vs = [rw_refs[b][pl.ds(ch * _H + ys_[b], 1), :]
                      for ch in range(4)]
                picks.append([jnp.sum(jnp.where(hits[b], v, 0.0))
                              for v in vs])

            for b in B:
                rx, ry, ww, wl = picks[b]
                m = ms[b]
                xs = xcols[b].astype(jnp.float32) + rx
                ys = ys_[b].astype(jnp.float32) + ry
                x1 = jnp.clip((xs - ww * 0.5) * _DOWN, 0.0, _IMG)
                y1 = jnp.clip((ys - wl * 0.5) * _DOWN, 0.0, _IMG)
                x2 = jnp.clip((xs + ww * 0.5) * _DOWN, 0.0, _IMG)
                y2 = jnp.clip((ys + wl * 0.5) * _DOWN, 0.0, _IMG)
                score = 1.0 / (1.0 + jnp.exp(-m))
                valid = (score >= _THRESH).astype(jnp.float32)
                rowvec = jnp.where(lane_i == 0, x1,
                          jnp.where(lane_i == 1, y1,
                           jnp.where(lane_i == 2, x2,
                            jnp.where(lane_i == 3, y2,
                             jnp.where(lane_i == 4, score,
                              jnp.where(lane_i == 5,
                                        cs[b].astype(jnp.float32), 0.0))))))
                o_ref[b, pl.ds(k, 1), :] = rowvec * valid
            return 0

        jax.lax.fori_loop(0, _K, step, 0)


def _wrapped(x):
    body = lambda x_hbm, o_ref, stage, sem, *rest: _body(
        x_hbm, o_ref, stage, sem,
        list(rest[:_B]), list(rest[_B:2 * _B]), list(rest[2 * _B:]))
    return pl.pallas_call(
        body,
        grid=(_B,),
        in_specs=[pl.BlockSpec(memory_space=pl.ANY)],
        out_specs=pl.BlockSpec((_B, 104, _W), lambda i: (0, 0, 0)),
        out_shape=jax.ShapeDtypeStruct((_B, 104, _W), jnp.float32),
        scratch_shapes=(
            [pltpu.VMEM((_C + 4, _H, _W), jnp.float32),
             pltpu.SemaphoreType.DMA]
            + [pltpu.VMEM((_C * _H, _W), jnp.float32) for _ in range(_B)]
            + [pltpu.VMEM((_C, _H), jnp.float32) for _ in range(_B)]
            + [pltpu.VMEM((4 * _H, _W), jnp.float32) for _ in range(_B)]
        ),
    )(x)


@jax.jit
def kernel(out_features):
    padded = _wrapped(out_features)
    return padded[:, :_K, :6]
